# baseline (speedup 1.0000x reference)
import jax
import jax.numpy as jnp
from jax.experimental import pallas as pl

N = 4096
FUSED = 1024
MLP_BLK = 512


def _mlp_body(x_ref, wa_ref, ba_ref, wb_ref, bb_ref, o_ref):
    h = jnp.maximum(
        jnp.dot(x_ref[0], wa_ref[0], preferred_element_type=jnp.float32)
        + ba_ref[0],
        0.0,
    )
    o_ref[0] = (
        jnp.dot(h, wb_ref[0], preferred_element_type=jnp.float32) + bb_ref[0]
    )


def _mlp_heads(x1, x2, W1a, b1a, W1b, b1b, W2a, b2a, W2b, b2b):
    H = W1a.shape[1]
    O = W1b.shape[1]
    xs = jnp.stack([x1, x2])
    Wa = jnp.stack([W1a, W2a])
    ba = jnp.stack([b1a.reshape(1, H), b2a.reshape(1, H)])
    Wb = jnp.stack([W1b, W2b])
    bb = jnp.stack([b1b.reshape(1, O), b2b.reshape(1, O)])
    out = pl.pallas_call(
        _mlp_body,
        grid=(2, N // MLP_BLK),
        in_specs=[
            pl.BlockSpec((1, MLP_BLK, FUSED), lambda c, i: (c, i, 0)),
            pl.BlockSpec((1, FUSED, H), lambda c, i: (c, 0, 0)),
            pl.BlockSpec((1, 1, H), lambda c, i: (c, 0, 0)),
            pl.BlockSpec((1, H, O), lambda c, i: (c, 0, 0)),
            pl.BlockSpec((1, 1, O), lambda c, i: (c, 0, 0)),
        ],
        out_specs=pl.BlockSpec((1, MLP_BLK, O), lambda c, i: (c, i, 0)),
        out_shape=jax.ShapeDtypeStruct((2, N, O), jnp.float32),
    )(xs, Wa, ba, Wb, bb)
    return out[0], out[1]


def _select_topk_by_entropy(x, fused_dim=FUSED):
    abs_x = jnp.abs(x)
    probs = abs_x / (abs_x.sum(axis=0, keepdims=True) + 1e-08)
    entropy = -(probs * jnp.log(probs + 1e-08)).sum(axis=0)
    _, topk_idx = jax.lax.top_k(entropy, fused_dim)
    return jnp.take(x, topk_idx, axis=1)


def kernel(a, b, W1a, b1a, W1b, b1b, W2a, b2a, W2b, b2b):
    S1 = jnp.concatenate([a, b], axis=1)
    fused1 = _select_topk_by_entropy(S1)
    S2 = jnp.concatenate([fused1, b], axis=1)
    fused2 = _select_topk_by_entropy(S2)
    logits1, logits2 = _mlp_heads(
        fused1, fused2, W1a, b1a, W1b, b1b, W2a, b2a, W2b, b2b
    )
    return (logits1, logits2, fused1, fused2)


# speedup vs baseline: 1.0675x; 1.0675x over previous
import jax
import jax.numpy as jnp
from jax.experimental import pallas as pl

N = 4096
FUSED = 1024
MLP_BLK = 512


def _mlp_body(xt_ref, wa_ref, ba_ref, wb_ref, bb_ref, o_ref):
    xb = xt_ref[...].astype(jnp.bfloat16)
    h = jnp.maximum(
        jax.lax.dot_general(
            xb, wa_ref[...],
            (((0,), (0,)), ((), ())),
            preferred_element_type=jnp.float32,
        )
        + ba_ref[...],
        0.0,
    ).astype(jnp.bfloat16)
    o_ref[...] = (
        jnp.dot(h, wb_ref[...], preferred_element_type=jnp.float32)
        + bb_ref[...]
    )


def _mlp_head(x, Wa, ba, Wb, bb):
    H = Wa.shape[1]
    O = Wb.shape[1]
    xt = jnp.transpose(x)
    return pl.pallas_call(
        _mlp_body,
        grid=(N // MLP_BLK,),
        in_specs=[
            pl.BlockSpec((FUSED, MLP_BLK), lambda i: (0, i)),
            pl.BlockSpec((FUSED, H), lambda i: (0, 0)),
            pl.BlockSpec((1, H), lambda i: (0, 0)),
            pl.BlockSpec((H, O), lambda i: (0, 0)),
            pl.BlockSpec((1, O), lambda i: (0, 0)),
        ],
        out_specs=pl.BlockSpec((MLP_BLK, O), lambda i: (i, 0)),
        out_shape=jax.ShapeDtypeStruct((N, O), jnp.float32),
    )(
        xt,
        Wa.astype(jnp.bfloat16),
        ba.reshape(1, H),
        Wb.astype(jnp.bfloat16),
        bb.reshape(1, O),
    )


def _select_topk_by_entropy(x, fused_dim=FUSED):
    abs_x = jnp.abs(x)
    probs = abs_x / (abs_x.sum(axis=0, keepdims=True) + 1e-08)
    entropy = -(probs * jnp.log(probs + 1e-08)).sum(axis=0)
    _, topk_idx = jax.lax.top_k(entropy, fused_dim)
    return jnp.take(x, topk_idx, axis=1)


def kernel(a, b, W1a, b1a, W1b, b1b, W2a, b2a, W2b, b2b):
    S1 = jnp.concatenate([a, b], axis=1)
    fused1 = _select_topk_by_entropy(S1)
    S2 = jnp.concatenate([fused1, b], axis=1)
    fused2 = _select_topk_by_entropy(S2)
    logits1 = _mlp_head(fused1, W1a, b1a, W1b, b1b)
    logits2 = _mlp_head(fused2, W2a, b2a, W2b, b2b)
    return (logits1, logits2, fused1, fused2)


# speedup vs baseline: 1.0716x; 1.0038x over previous
import jax
import jax.numpy as jnp
from jax.experimental import pallas as pl

N = 4096
FUSED = 1024
MLP_BLK = 512


def _mlp_body(x_ref, wa_ref, ba_ref, wb_ref, bb_ref, o_ref):
    xb = x_ref[...].astype(jnp.bfloat16)
    h = jnp.maximum(
        jnp.dot(xb, wa_ref[...], preferred_element_type=jnp.float32)
        + ba_ref[...],
        0.0,
    ).astype(jnp.bfloat16)
    o_ref[...] = (
        jnp.dot(h, wb_ref[...], preferred_element_type=jnp.float32)
        + bb_ref[...]
    )


def _mlp_head(x, Wa, ba, Wb, bb):
    H = Wa.shape[1]
    O = Wb.shape[1]
    return pl.pallas_call(
        _mlp_body,
        grid=(N // MLP_BLK,),
        in_specs=[
            pl.BlockSpec((MLP_BLK, FUSED), lambda i: (i, 0)),
            pl.BlockSpec((FUSED, H), lambda i: (0, 0)),
            pl.BlockSpec((1, H), lambda i: (0, 0)),
            pl.BlockSpec((H, O), lambda i: (0, 0)),
            pl.BlockSpec((1, O), lambda i: (0, 0)),
        ],
        out_specs=pl.BlockSpec((MLP_BLK, O), lambda i: (i, 0)),
        out_shape=jax.ShapeDtypeStruct((N, O), jnp.float32),
    )(
        x,
        Wa.astype(jnp.bfloat16),
        ba.reshape(1, H),
        Wb.astype(jnp.bfloat16),
        bb.reshape(1, O),
    )


def _select_topk_by_entropy(x, fused_dim=FUSED):
    abs_x = jnp.abs(x)
    probs = abs_x / (abs_x.sum(axis=0, keepdims=True) + 1e-08)
    entropy = -(probs * jnp.log(probs + 1e-08)).sum(axis=0)
    _, topk_idx = jax.lax.top_k(entropy, fused_dim)
    return jnp.take(x, topk_idx, axis=1)


def kernel(a, b, W1a, b1a, W1b, b1b, W2a, b2a, W2b, b2b):
    S1 = jnp.concatenate([a, b], axis=1)
    fused1 = _select_topk_by_entropy(S1)
    S2 = jnp.concatenate([fused1, b], axis=1)
    fused2 = _select_topk_by_entropy(S2)
    logits1 = _mlp_head(fused1, W1a, b1a, W1b, b1b)
    logits2 = _mlp_head(fused2, W2a, b2a, W2b, b2b)
    return (logits1, logits2, fused1, fused2)


# speedup vs baseline: 1.0793x; 1.0072x over previous
import jax
import jax.numpy as jnp
from jax.experimental import pallas as pl

N = 4096
FUSED = 1024
MLP_BLK = 512


def _mlp_body(
    x1_ref, x2_ref, wa1_ref, ba1_ref, wb1_ref, bb1_ref,
    wa2_ref, ba2_ref, wb2_ref, bb2_ref, o1_ref, o2_ref,
):
    for x_ref, wa, ba, wb, bb, o_ref in (
        (x1_ref, wa1_ref, ba1_ref, wb1_ref, bb1_ref, o1_ref),
        (x2_ref, wa2_ref, ba2_ref, wb2_ref, bb2_ref, o2_ref),
    ):
        xb = x_ref[...].astype(jnp.bfloat16)
        h = jnp.maximum(
            jnp.dot(xb, wa[...], preferred_element_type=jnp.float32) + ba[...],
            0.0,
        ).astype(jnp.bfloat16)
        o_ref[...] = (
            jnp.dot(h, wb[...], preferred_element_type=jnp.float32) + bb[...]
        )


def _mlp_heads(x1, x2, W1a, b1a, W1b, b1b, W2a, b2a, W2b, b2b):
    H = W1a.shape[1]
    O = W1b.shape[1]
    xspec = pl.BlockSpec((MLP_BLK, FUSED), lambda i: (i, 0))
    wa_spec = pl.BlockSpec((FUSED, H), lambda i: (0, 0))
    ba_spec = pl.BlockSpec((1, H), lambda i: (0, 0))
    wb_spec = pl.BlockSpec((H, O), lambda i: (0, 0))
    bb_spec = pl.BlockSpec((1, O), lambda i: (0, 0))
    ospec = pl.BlockSpec((MLP_BLK, O), lambda i: (i, 0))
    out_sh = jax.ShapeDtypeStruct((N, O), jnp.float32)
    return pl.pallas_call(
        _mlp_body,
        grid=(N // MLP_BLK,),
        in_specs=[xspec, xspec, wa_spec, ba_spec, wb_spec, bb_spec,
                  wa_spec, ba_spec, wb_spec, bb_spec],
        out_specs=[ospec, ospec],
        out_shape=[out_sh, out_sh],
    )(
        x1, x2,
        W1a.astype(jnp.bfloat16), b1a.reshape(1, H),
        W1b.astype(jnp.bfloat16), b1b.reshape(1, O),
        W2a.astype(jnp.bfloat16), b2a.reshape(1, H),
        W2b.astype(jnp.bfloat16), b2b.reshape(1, O),
    )


def _select_topk_by_entropy(x, fused_dim=FUSED):
    abs_x = jnp.abs(x)
    probs = abs_x / (abs_x.sum(axis=0, keepdims=True) + 1e-08)
    entropy = -(probs * jnp.log(probs + 1e-08)).sum(axis=0)
    _, topk_idx = jax.lax.top_k(entropy, fused_dim)
    return jnp.take(x, topk_idx, axis=1)


def kernel(a, b, W1a, b1a, W1b, b1b, W2a, b2a, W2b, b2b):
    S1 = jnp.concatenate([a, b], axis=1)
    fused1 = _select_topk_by_entropy(S1)
    S2 = jnp.concatenate([fused1, b], axis=1)
    fused2 = _select_topk_by_entropy(S2)
    logits1, logits2 = _mlp_heads(
        fused1, fused2, W1a, b1a, W1b, b1b, W2a, b2a, W2b, b2b
    )
    return (logits1, logits2, fused1, fused2)
